# baseline (speedup 1.0000x reference)
import functools

import jax
import jax.numpy as jnp
from jax import lax
from jax.experimental import pallas as pl
from jax.experimental.pallas import tpu as pltpu
from jax.experimental.pallas import tpu_sc as plsc

_NUM_CLASS = 19
_MARGIN = 0.5
_FACTOR = 1.0

_NC = 2
_NS = 16
_NW = _NC * _NS
_L = 16

_C = 96
_H = 512
_W = 512
_RPW = _H // _NS
_PPW = _RPW * _W
_KPAD = _NUM_CLASS * _L
_ACC = _C * _KPAD
_JV = _W // _L


_CG = 4
_NG = _C // _CG
_RH = _RPW // 2


def _sc_body(feat_hbm, lab_hbm, psum_hbm, pcnt_hbm,
             lbuf, fbuf0, fbuf1, acc, cnt, sem0, sem1):
    wid = lax.axis_index("c") * _NS + lax.axis_index("s")
    b = wid // _NS
    h0 = (wid % _NS) * _RPW

    iota = lax.iota(jnp.int32, _L)
    ones = jnp.ones((_L,), jnp.float32)
    zeros = jnp.zeros((_L,), jnp.float32)

    pltpu.sync_copy(lab_hbm.at[pl.ds(b * _H + h0, _RPW), :], lbuf)

    def _feat_src(g, half):
        return feat_hbm.at[pl.ds(b * _C + g * _CG, _CG),
                           pl.ds(h0 + half * _RH, _RH), :]

    pltpu.make_async_copy(_feat_src(0, 0), fbuf0, sem0).start()
    pltpu.make_async_copy(_feat_src(0, 1), fbuf1, sem1).start()

    @plsc.parallel_loop(0, _ACC // _L, unroll=8)
    def _zacc(i):
        acc[pl.ds(i * _L, _L)] = zeros

    @plsc.parallel_loop(0, _KPAD // _L, unroll=1)
    def _zcnt(i):
        cnt[pl.ds(i * _L, _L)] = zeros

    @plsc.parallel_loop(0, _RPW, unroll=2)
    def _cbody(i):
        for j in range(_JV):
            lv = lbuf[i, pl.ds(j * _L, _L)]
            plsc.addupdate_scatter(cnt, [lv * _L + iota], ones)

    def _chunk(g, carry):
        bases = [iota + (g * _CG + k) * _KPAD for k in range(_CG)]
        for fb, sem, half in ((fbuf0, sem0, 0), (fbuf1, sem1, 1)):
            pltpu.make_async_copy(_feat_src(0, 0), fb, sem).wait()

            @plsc.parallel_loop(0, _RH, unroll=2)
            def _inner(i):
                for j in range(_JV):
                    lv = lbuf[half * _RH + i, pl.ds(j * _L, _L)] * _L
                    for k in range(_CG):
                        v = fb[k, i, pl.ds(j * _L, _L)]
                        plsc.addupdate_scatter(acc, [lv + bases[k]], v)

            @pl.when(g < _NG - 1)
            def _prefetch():
                pltpu.make_async_copy(_feat_src(g + 1, half), fb, sem).start()
        return carry

    lax.fori_loop(0, _NG, _chunk, 0)

    pltpu.sync_copy(acc, psum_hbm.at[pl.ds(wid * _ACC, _ACC)])
    pltpu.sync_copy(cnt, pcnt_hbm.at[pl.ds(wid * _KPAD, _KPAD)])


_sc_segsum = functools.partial(
    pl.kernel,
    out_type=(
        jax.ShapeDtypeStruct((_NW * _ACC,), jnp.float32),
        jax.ShapeDtypeStruct((_NW * _KPAD,), jnp.float32),
    ),
    mesh=plsc.VectorSubcoreMesh(
        core_axis_name="c", subcore_axis_name="s",
        num_cores=_NC, num_subcores=_NS),
    compiler_params=pltpu.CompilerParams(
        needs_layout_passes=False, use_tc_tiling_on_sc=True),
    scratch_types=[
        pltpu.VMEM((_RPW, _W), jnp.int32),
        pltpu.VMEM((_CG, _RH, _W), jnp.float32),
        pltpu.VMEM((_CG, _RH, _W), jnp.float32),
        pltpu.VMEM((_ACC,), jnp.float32),
        pltpu.VMEM((_KPAD,), jnp.float32),
        pltpu.SemaphoreType.DMA,
        pltpu.SemaphoreType.DMA,
    ],
)(_sc_body)


def _loss_body(psum_ref, pcnt_ref, out_ref):
    s4 = psum_ref[...]
    c3 = pcnt_ref[...]
    s = jnp.sum(jnp.sum(s4, axis=3), axis=0)
    cnt = jnp.sum(jnp.sum(c3, axis=2), axis=0, keepdims=True)
    m = jnp.where(cnt > 0.0, s / jnp.maximum(cnt, 1.0), 0.0)
    n2 = jnp.sum(m * m, axis=0, keepdims=True)
    fn = m / jnp.maximum(jnp.sqrt(n2), 1e-12)
    g = lax.dot_general(fn, fn, (((0,), (0,)), ((), ())),
                        preferred_element_type=jnp.float32)
    d = 1.0 - g
    ii = lax.broadcasted_iota(jnp.int32, (_NUM_CLASS, _NUM_CLASS), 0)
    jj = lax.broadcasted_iota(jnp.int32, (_NUM_CLASS, _NUM_CLASS), 1)
    d = jnp.where(ii == jj, 2.0, d)
    presentf = jnp.where(cnt > 0.0, 1.0, 0.0)
    pair = lax.dot_general(presentf, presentf, (((0,), (0,)), ((), ())),
                           preferred_element_type=jnp.float32)
    vals = pair * jnp.maximum(_MARGIN - d, 0.0)
    n = jnp.sum(presentf)
    out_ref[...] = jnp.reshape(_FACTOR * jnp.sum(vals) / (n * n), (1, 1))


_loss_tc = pl.pallas_call(
    _loss_body,
    out_shape=jax.ShapeDtypeStruct((1, 1), jnp.float32),
)


def kernel(features, labels, prototypes):
    del prototypes
    feat2 = features.reshape(2 * _C, _H, _W)
    lab2 = labels.reshape(2 * _H, _W)
    psum, pcnt = _sc_segsum(feat2, lab2)
    loss = _loss_tc(psum.reshape(_NW, _C, _NUM_CLASS, _L),
                    pcnt.reshape(_NW, _NUM_CLASS, _L))
    return loss[0, 0]


# speedup vs baseline: 1.1019x; 1.1019x over previous
import functools

import jax
import jax.numpy as jnp
from jax import lax
from jax.experimental import pallas as pl
from jax.experimental.pallas import tpu as pltpu
from jax.experimental.pallas import tpu_sc as plsc

_NUM_CLASS = 19
_MARGIN = 0.5
_FACTOR = 1.0

_NC = 2
_NS = 16
_NW = _NC * _NS
_L = 16

_C = 96
_H = 512
_W = 512
_RPW = _H // _NS
_PPW = _RPW * _W
_KPAD = _NUM_CLASS * _L
_ACC = _C * _KPAD
_JV = _W // _L


_CG = 4
_NG = _C // _CG
_RH = _RPW // 2


def _sc_body(feat_hbm, lab_hbm, psum_hbm, pcnt_hbm,
             lbuf, fbuf0, fbuf1, acc, cnt, sem0, sem1):
    wid = lax.axis_index("c") * _NS + lax.axis_index("s")
    b = wid // _NS
    h0 = (wid % _NS) * _RPW

    iota = lax.iota(jnp.int32, _L)
    ones = jnp.ones((_L,), jnp.float32)
    zeros = jnp.zeros((_L,), jnp.float32)

    pltpu.sync_copy(lab_hbm.at[pl.ds(b * _H + h0, _RPW), :], lbuf)

    def _feat_src(c):
        row = (b * _C + c) * _H + h0
        return feat_hbm.at[pl.ds(row, _RPW), :]

    pltpu.make_async_copy(_feat_src(0), fbuf0, sem0).start()
    pltpu.make_async_copy(_feat_src(1), fbuf1, sem1).start()

    @plsc.parallel_loop(0, _ACC // _L, unroll=8)
    def _zacc(i):
        acc[pl.ds(i * _L, _L)] = zeros

    @plsc.parallel_loop(0, _KPAD // _L, unroll=1)
    def _zcnt(i):
        cnt[pl.ds(i * _L, _L)] = zeros

    @plsc.parallel_loop(0, _RPW, unroll=2)
    def _cbody(i):
        for j in range(_JV):
            lv = lbuf[i, pl.ds(j * _L, _L)]
            plsc.addupdate_scatter(cnt, [lv * _L + iota], ones)

    def _chan(cb, carry):
        for fb, sem, par in ((fbuf0, sem0, 0), (fbuf1, sem1, 1)):
            c = cb * 2 + par
            pltpu.make_async_copy(_feat_src(0), fb, sem).wait()
            base = iota + c * _KPAD

            @plsc.parallel_loop(0, _RPW, unroll=4)
            def _inner(i):
                for j in range(_JV):
                    lv = lbuf[i, pl.ds(j * _L, _L)]
                    v = fb[i, pl.ds(j * _L, _L)]
                    plsc.addupdate_scatter(acc, [lv * _L + base], v)

            @pl.when(cb < _C // 2 - 1)
            def _prefetch():
                pltpu.make_async_copy(_feat_src(c + 2), fb, sem).start()
        return carry

    lax.fori_loop(0, _C // 2, _chan, 0)

    pltpu.sync_copy(acc, psum_hbm.at[pl.ds(wid * _ACC, _ACC)])
    pltpu.sync_copy(cnt, pcnt_hbm.at[pl.ds(wid * _KPAD, _KPAD)])


_sc_segsum = functools.partial(
    pl.kernel,
    out_type=(
        jax.ShapeDtypeStruct((_NW * _ACC,), jnp.float32),
        jax.ShapeDtypeStruct((_NW * _KPAD,), jnp.float32),
    ),
    mesh=plsc.VectorSubcoreMesh(
        core_axis_name="c", subcore_axis_name="s",
        num_cores=_NC, num_subcores=_NS),
    compiler_params=pltpu.CompilerParams(
        needs_layout_passes=False, use_tc_tiling_on_sc=True),
    scratch_types=[
        pltpu.VMEM((_RPW, _W), jnp.int32),
        pltpu.VMEM((_RPW, _W), jnp.float32),
        pltpu.VMEM((_RPW, _W), jnp.float32),
        pltpu.VMEM((_ACC,), jnp.float32),
        pltpu.VMEM((_KPAD,), jnp.float32),
        pltpu.SemaphoreType.DMA,
        pltpu.SemaphoreType.DMA,
    ],
)(_sc_body)


def _loss_body(psum_ref, pcnt_ref, out_ref):
    s4 = psum_ref[...]
    c3 = pcnt_ref[...]
    s = jnp.sum(jnp.sum(s4, axis=3), axis=0)
    cnt = jnp.sum(jnp.sum(c3, axis=2), axis=0, keepdims=True)
    m = jnp.where(cnt > 0.0, s / jnp.maximum(cnt, 1.0), 0.0)
    n2 = jnp.sum(m * m, axis=0, keepdims=True)
    fn = m / jnp.maximum(jnp.sqrt(n2), 1e-12)
    g = lax.dot_general(fn, fn, (((0,), (0,)), ((), ())),
                        preferred_element_type=jnp.float32)
    d = 1.0 - g
    ii = lax.broadcasted_iota(jnp.int32, (_NUM_CLASS, _NUM_CLASS), 0)
    jj = lax.broadcasted_iota(jnp.int32, (_NUM_CLASS, _NUM_CLASS), 1)
    d = jnp.where(ii == jj, 2.0, d)
    presentf = jnp.where(cnt > 0.0, 1.0, 0.0)
    pair = lax.dot_general(presentf, presentf, (((0,), (0,)), ((), ())),
                           preferred_element_type=jnp.float32)
    vals = pair * jnp.maximum(_MARGIN - d, 0.0)
    n = jnp.sum(presentf)
    out_ref[...] = jnp.reshape(_FACTOR * jnp.sum(vals) / (n * n), (1, 1))


_loss_tc = pl.pallas_call(
    _loss_body,
    out_shape=jax.ShapeDtypeStruct((1, 1), jnp.float32),
)


def kernel(features, labels, prototypes):
    del prototypes
    feat2 = features.reshape(2 * _C * _H, _W)
    lab2 = labels.reshape(2 * _H, _W)
    psum, pcnt = _sc_segsum(feat2, lab2)
    loss = _loss_tc(psum.reshape(_NW, _C, _NUM_CLASS, _L),
                    pcnt.reshape(_NW, _NUM_CLASS, _L))
    return loss[0, 0]


# speedup vs baseline: 1.3998x; 1.2704x over previous
import functools

import jax
import jax.numpy as jnp
from jax import lax
from jax.experimental import pallas as pl
from jax.experimental.pallas import tpu as pltpu
from jax.experimental.pallas import tpu_sc as plsc

_NUM_CLASS = 19
_MARGIN = 0.5
_FACTOR = 1.0

_NC = 2
_NS = 16
_NW = _NC * _NS
_L = 16

_C = 96
_H = 512
_W = 512
_RPW = _H // _NS
_PPW = _RPW * _W
_KPAD = _NUM_CLASS * _L
_ACC = _C * _KPAD
_JV = _W // _L


_CG = 4
_NG = _C // _CG
_RH = _RPW // 2


def _sc_body(feat_hbm, lab_hbm, psum_hbm, pcnt_hbm,
             lbuf, fbuf0, fbuf1, acc, cnt, sem0, sem1):
    wid = lax.axis_index("c") * _NS + lax.axis_index("s")
    b = wid // _NS
    h0 = (wid % _NS) * _RPW

    iota = lax.iota(jnp.int32, _L)
    ones = jnp.ones((_L,), jnp.float32)
    zeros = jnp.zeros((_L,), jnp.float32)

    pltpu.sync_copy(lab_hbm.at[pl.ds(b * _H + h0, _RPW), :], lbuf)

    def _feat_src(c):
        row = (b * _C + c) * _H + h0
        return feat_hbm.at[pl.ds(row, _RPW), :]

    pltpu.make_async_copy(_feat_src(0), fbuf0, sem0).start()
    pltpu.make_async_copy(_feat_src(1), fbuf1, sem1).start()

    @plsc.parallel_loop(0, _ACC // _L, unroll=8)
    def _zacc(i):
        acc[pl.ds(i * _L, _L)] = zeros

    @plsc.parallel_loop(0, _KPAD // _L, unroll=1)
    def _zcnt(i):
        cnt[pl.ds(i * _L, _L)] = zeros

    @plsc.parallel_loop(0, _RPW, unroll=2)
    def _cbody(i):
        for j in range(_JV):
            lv = lbuf[i, pl.ds(j * _L, _L)]
            plsc.addupdate_scatter(cnt, [lv * _L + iota], ones)

    def _chan(cb, carry):
        for fb, sem, par in ((fbuf0, sem0, 0), (fbuf1, sem1, 1)):
            c = cb * 2 + par
            pltpu.make_async_copy(_feat_src(0), fb, sem).wait()
            base = iota + c * _KPAD

            @plsc.parallel_loop(0, _RPW * _JV, unroll=8)
            def _inner(t):
                i = t // _JV
                j = t % _JV
                lv = lbuf[i, pl.ds(j * _L, _L)]
                v = fb[i, pl.ds(j * _L, _L)]
                plsc.addupdate_scatter(acc, [lv * _L + base], v)

            @pl.when(cb < _C // 2 - 1)
            def _prefetch():
                pltpu.make_async_copy(_feat_src(c + 2), fb, sem).start()
        return carry

    lax.fori_loop(0, _C // 2, _chan, 0)

    pltpu.sync_copy(acc, psum_hbm.at[pl.ds(wid * _ACC, _ACC)])
    pltpu.sync_copy(cnt, pcnt_hbm.at[pl.ds(wid * _KPAD, _KPAD)])


_sc_segsum = functools.partial(
    pl.kernel,
    out_type=(
        jax.ShapeDtypeStruct((_NW * _ACC,), jnp.float32),
        jax.ShapeDtypeStruct((_NW * _KPAD,), jnp.float32),
    ),
    mesh=plsc.VectorSubcoreMesh(
        core_axis_name="c", subcore_axis_name="s",
        num_cores=_NC, num_subcores=_NS),
    compiler_params=pltpu.CompilerParams(
        needs_layout_passes=False, use_tc_tiling_on_sc=True),
    scratch_types=[
        pltpu.VMEM((_RPW, _W), jnp.int32),
        pltpu.VMEM((_RPW, _W), jnp.float32),
        pltpu.VMEM((_RPW, _W), jnp.float32),
        pltpu.VMEM((_ACC,), jnp.float32),
        pltpu.VMEM((_KPAD,), jnp.float32),
        pltpu.SemaphoreType.DMA,
        pltpu.SemaphoreType.DMA,
    ],
)(_sc_body)


def _loss_body(psum_ref, pcnt_ref, out_ref):
    s4 = psum_ref[...]
    c3 = pcnt_ref[...]
    s = jnp.sum(jnp.sum(s4, axis=3), axis=0)
    cnt = jnp.sum(jnp.sum(c3, axis=2), axis=0, keepdims=True)
    m = jnp.where(cnt > 0.0, s / jnp.maximum(cnt, 1.0), 0.0)
    n2 = jnp.sum(m * m, axis=0, keepdims=True)
    fn = m / jnp.maximum(jnp.sqrt(n2), 1e-12)
    g = lax.dot_general(fn, fn, (((0,), (0,)), ((), ())),
                        preferred_element_type=jnp.float32)
    d = 1.0 - g
    ii = lax.broadcasted_iota(jnp.int32, (_NUM_CLASS, _NUM_CLASS), 0)
    jj = lax.broadcasted_iota(jnp.int32, (_NUM_CLASS, _NUM_CLASS), 1)
    d = jnp.where(ii == jj, 2.0, d)
    presentf = jnp.where(cnt > 0.0, 1.0, 0.0)
    pair = lax.dot_general(presentf, presentf, (((0,), (0,)), ((), ())),
                           preferred_element_type=jnp.float32)
    vals = pair * jnp.maximum(_MARGIN - d, 0.0)
    n = jnp.sum(presentf)
    out_ref[...] = jnp.reshape(_FACTOR * jnp.sum(vals) / (n * n), (1, 1))


_loss_tc = pl.pallas_call(
    _loss_body,
    out_shape=jax.ShapeDtypeStruct((1, 1), jnp.float32),
)


def kernel(features, labels, prototypes):
    del prototypes
    feat2 = features.reshape(2 * _C * _H, _W)
    lab2 = labels.reshape(2 * _H, _W)
    psum, pcnt = _sc_segsum(feat2, lab2)
    loss = _loss_tc(psum.reshape(_NW, _C, _NUM_CLASS, _L),
                    pcnt.reshape(_NW, _NUM_CLASS, _L))
    return loss[0, 0]
